# Initial kernel scaffold; baseline (speedup 1.0000x reference)
import jax
import jax.numpy as jnp
from jax.experimental import pallas as pl


def kernel(x, edge_index, W1l, b1l, W1r, W2l, b2l, W2r):
    raise NotImplementedError("write your pallas kernel here")


# speedup vs baseline: 5.4296x; 5.4296x over previous
import functools

import jax
import jax.numpy as jnp
from jax import lax
from jax.experimental import pallas as pl
from jax.experimental.pallas import tpu as pltpu
from jax.experimental.pallas import tpu_sc as plsc

N_WORKERS = 32
CHUNK = 80


def _make_sc_aggregate(n, d, e, with_cnt):
    epw = e // N_WORKERS
    steps = epw // CHUNK
    assert epw % CHUNK == 0, (e, epw)
    rpt = n // 16
    assert n % 16 == 0

    mesh = plsc.VectorSubcoreMesh(core_axis_name="c", subcore_axis_name="s")

    out_type = [jax.ShapeDtypeStruct((2, n, d), jnp.float32)]
    if with_cnt:
        out_type.append(jax.ShapeDtypeStruct((2, n), jnp.float32))

    scratch = [
        pltpu.VMEM((CHUNK,), jnp.int32),
        pltpu.VMEM((CHUNK,), jnp.int32),
        pltpu.VMEM((CHUNK, d), jnp.float32),
        pltpu.VMEM_SHARED((n, d), jnp.float32),
        pltpu.SemaphoreType.DMA,
    ]
    if with_cnt:
        scratch += [
            pltpu.VMEM((CHUNK,), jnp.float32),
            pltpu.VMEM_SHARED((n,), jnp.float32),
        ]

    def body(x_hbm, src_hbm, dst_hbm, z2_hbm, z1_hbm, ones_hbm,
             agg_out, *rest):
        if with_cnt:
            cnt_out, srcb, dstb, rowsb, aggs, sem, onesb, cnts = rest
        else:
            srcb, dstb, rowsb, aggs, sem = rest
        c = lax.axis_index("c")
        s = lax.axis_index("s")
        wid = c * 16 + s

        r0 = pl.multiple_of(s * rpt, 8)
        pltpu.sync_copy(z2_hbm.at[pl.ds(r0, rpt)], aggs.at[pl.ds(r0, rpt)])
        if with_cnt:
            @pl.when(s == 0)
            def _():
                pltpu.sync_copy(z1_hbm, cnts)
            pltpu.sync_copy(ones_hbm, onesb)
        plsc.subcore_barrier()

        def step(i, carry):
            base = pl.multiple_of(wid * epw + i * CHUNK, 8)
            pltpu.sync_copy(src_hbm.at[pl.ds(base, CHUNK)], srcb)
            pltpu.sync_copy(dst_hbm.at[pl.ds(base, CHUNK)], dstb)
            pltpu.async_copy(x_hbm.at[srcb], rowsb, sem).wait()
            pltpu.sync_copy(rowsb, aggs.at[dstb], add=True)
            if with_cnt:
                pltpu.sync_copy(onesb, cnts.at[dstb], add=True)
            return carry

        lax.fori_loop(0, steps, step, 0)
        plsc.subcore_barrier()

        pltpu.sync_copy(aggs.at[pl.ds(r0, rpt)], agg_out.at[c, pl.ds(r0, rpt)])
        if with_cnt:
            @pl.when(s == 0)
            def _():
                pltpu.sync_copy(cnts, cnt_out.at[c])

    return pl.kernel(body, out_type=out_type, mesh=mesh,
                     scratch_types=scratch,
                     compiler_params=pltpu.CompilerParams(
                         use_tc_tiling_on_sc=False))


def _make_tc_layer(n, d, relu, block_rows=2000):
    assert n % block_rows == 0

    def body(aa, ab, ca, cb, xr, wl, bl, wr, o):
        cnt = jnp.maximum(ca[...] + cb[...], 1.0)
        mean = (aa[...] + ab[...]) / cnt
        acc = lax.dot_general(mean, wl[...], (((1,), (1,)), ((), ())),
                              preferred_element_type=jnp.float32)
        acc = acc + lax.dot_general(xr[...], wr[...], (((1,), (1,)), ((), ())),
                                    preferred_element_type=jnp.float32)
        acc = acc + bl[...]
        o[...] = jnp.maximum(acc, 0.0) if relu else acc

    rows = pl.BlockSpec((block_rows, d), lambda i: (i, 0))
    return pl.pallas_call(
        body,
        grid=(n // block_rows,),
        in_specs=[
            rows, rows,
            pl.BlockSpec((block_rows, 1), lambda i: (i, 0)),
            pl.BlockSpec((block_rows, 1), lambda i: (i, 0)),
            rows,
            pl.BlockSpec((d, d), lambda i: (0, 0)),
            pl.BlockSpec((1, d), lambda i: (0, 0)),
            pl.BlockSpec((d, d), lambda i: (0, 0)),
        ],
        out_specs=rows,
        out_shape=jax.ShapeDtypeStruct((n, d), jnp.float32),
    )


def kernel(x, edge_index, W1l, b1l, W1r, W2l, b2l, W2r):
    n, d = x.shape
    e = edge_index.shape[1]
    ei = edge_index.astype(jnp.int32)
    src, dst = ei[0], ei[1]

    z2 = jnp.zeros((n, d), jnp.float32)
    z1 = jnp.zeros((n,), jnp.float32)
    ones = jnp.ones((CHUNK,), jnp.float32)

    agg_cnt = _make_sc_aggregate(n, d, e, with_cnt=True)
    agg_only = _make_sc_aggregate(n, d, e, with_cnt=False)
    layer1 = _make_tc_layer(n, d, relu=True)
    layer2 = _make_tc_layer(n, d, relu=False)

    aggp, cntp = agg_cnt(x, src, dst, z2, z1, ones)
    ca = cntp[0].reshape(n, 1)
    cb = cntp[1].reshape(n, 1)
    b1 = b1l.reshape(1, d)
    b2 = b2l.reshape(1, d)

    h = layer1(aggp[0], aggp[1], ca, cb, x, W1l, b1, W1r)
    (aggp2,) = agg_only(h, src, dst, z2, z1, ones)
    out = layer2(aggp2[0], aggp2[1], ca, cb, h, W2l, b2, W2r)
    return out
